# Initial kernel scaffold; baseline (speedup 1.0000x reference)
import jax
import jax.numpy as jnp
from jax.experimental import pallas as pl


def kernel(inputs, embedding):
    raise NotImplementedError("write your pallas kernel here")


# speedup vs baseline: 1.4658x; 1.4658x over previous
import functools

import jax
import jax.numpy as jnp
from jax.experimental import pallas as pl
from jax.experimental.pallas import tpu as pltpu

_NUM_EMBEDDINGS = 1024
_EMBEDDING_DIM = 64
_BLOCK = 1024


def _vq_kernel(n_tokens, grid, x_ref, emb_ref, q_ref, idx_ref, loss_ref,
               perp_ref, counts_scr, loss_scr):
    i = pl.program_id(0)
    x = x_ref[...]
    emb = emb_ref[...]
    e2 = jnp.sum(emb * emb, axis=1)
    scores = -2.0 * jnp.dot(x, emb.T, preferred_element_type=jnp.float32)
    scores = scores + e2[None, :]
    idx = jnp.argmin(scores, axis=1).astype(jnp.int32)
    onehot = (jax.lax.broadcasted_iota(jnp.int32, (_BLOCK, _NUM_EMBEDDINGS), 1)
              == idx[:, None]).astype(jnp.float32)
    q = jnp.dot(onehot, emb, preferred_element_type=jnp.float32)
    q_ref[...] = x + (q - x)
    idx_ref[0, 0, :] = idx

    diff = q - x
    part_loss = jnp.sum(diff * diff)
    part_counts = jnp.sum(onehot, axis=0)

    @pl.when(i == 0)
    def _init():
        loss_scr[0, 0] = 0.0
        counts_scr[...] = jnp.zeros_like(counts_scr)

    loss_scr[0, 0] += part_loss
    counts_scr[...] += part_counts[None, :]

    @pl.when(i == grid - 1)
    def _finalize():
        loss_ref[0, 0] = loss_scr[0, 0] / (n_tokens * _EMBEDDING_DIM)
        p = counts_scr[0, :] * (1.0 / n_tokens)
        perp_ref[0, 0] = jnp.exp(-jnp.sum(p * jnp.log(p + 1e-10)))


def kernel(inputs, embedding):
    input_shape = inputs.shape
    flat = inputs.reshape(-1, _EMBEDDING_DIM)
    n_tokens = flat.shape[0]
    grid = n_tokens // _BLOCK

    quantized, idx3, loss, perp = pl.pallas_call(
        functools.partial(_vq_kernel, n_tokens, grid),
        grid=(grid,),
        in_specs=[
            pl.BlockSpec((_BLOCK, _EMBEDDING_DIM), lambda i: (i, 0)),
            pl.BlockSpec((_NUM_EMBEDDINGS, _EMBEDDING_DIM), lambda i: (0, 0)),
        ],
        out_specs=[
            pl.BlockSpec((_BLOCK, _EMBEDDING_DIM), lambda i: (i, 0)),
            pl.BlockSpec((1, 1, _BLOCK), lambda i: (i, 0, 0)),
            pl.BlockSpec(memory_space=pltpu.SMEM, block_shape=(1, 1),
                         index_map=lambda i: (0, 0)),
            pl.BlockSpec(memory_space=pltpu.SMEM, block_shape=(1, 1),
                         index_map=lambda i: (0, 0)),
        ],
        out_shape=[
            jax.ShapeDtypeStruct((n_tokens, _EMBEDDING_DIM), jnp.float32),
            jax.ShapeDtypeStruct((grid, 1, _BLOCK), jnp.int32),
            jax.ShapeDtypeStruct((1, 1), jnp.float32),
            jax.ShapeDtypeStruct((1, 1), jnp.float32),
        ],
        scratch_shapes=[
            pltpu.VMEM((1, _NUM_EMBEDDINGS), jnp.float32),
            pltpu.SMEM((1, 1), jnp.float32),
        ],
    )(flat, embedding)

    quantized = quantized.reshape(input_shape)
    indices = idx3.reshape(input_shape[:-1])
    return (quantized, loss.reshape(()), indices, perp.reshape(()))


# speedup vs baseline: 1.9322x; 1.3181x over previous
import functools

import jax
import jax.numpy as jnp
from jax.experimental import pallas as pl
from jax.experimental.pallas import tpu as pltpu

_NUM_EMBEDDINGS = 1024
_EMBEDDING_DIM = 64
_BLOCK = 1024


def _vq_kernel(n_tokens, grid, x_ref, emb_ref, q_ref, idx_ref, loss_ref,
               perp_ref, counts_scr, loss_scr):
    i = pl.program_id(0)
    x = x_ref[...]
    emb = emb_ref[...]
    e2 = jnp.dot(emb * emb, jnp.ones((_EMBEDDING_DIM, 1), jnp.float32),
                 preferred_element_type=jnp.float32)
    emb_aug = jnp.concatenate([emb * -2.0, e2], axis=1)
    ones_col = jnp.ones((_BLOCK, 1), jnp.float32)
    x_aug = jnp.concatenate([x, ones_col], axis=1)
    scores_t = jax.lax.dot_general(
        emb_aug, x_aug, (((1,), (1,)), ((), ())),
        preferred_element_type=jnp.float32)
    idx = jnp.argmin(scores_t, axis=0).astype(jnp.int32)
    onehot_t = (jax.lax.broadcasted_iota(jnp.int32, (_NUM_EMBEDDINGS, _BLOCK), 0)
                == idx[None, :]).astype(jnp.float32)
    q = jax.lax.dot_general(
        onehot_t, emb, (((0,), (0,)), ((), ())),
        preferred_element_type=jnp.float32)
    q_ref[...] = x + (q - x)
    idx_ref[0, 0, :] = idx

    diff = q - x
    part_loss = jnp.sum(diff * diff)
    ones_row = jnp.ones((1, _BLOCK), jnp.float32)
    part_counts = jax.lax.dot_general(
        ones_row, onehot_t, (((1,), (1,)), ((), ())),
        preferred_element_type=jnp.float32)

    @pl.when(i == 0)
    def _init():
        loss_scr[0, 0] = 0.0
        counts_scr[...] = jnp.zeros_like(counts_scr)

    loss_scr[0, 0] += part_loss
    counts_scr[...] += part_counts

    @pl.when(i == grid - 1)
    def _finalize():
        loss_ref[0, 0] = loss_scr[0, 0] / (n_tokens * _EMBEDDING_DIM)
        p = counts_scr[0, :] * (1.0 / n_tokens)
        perp_ref[0, 0] = jnp.exp(-jnp.sum(p * jnp.log(p + 1e-10)))


def kernel(inputs, embedding):
    input_shape = inputs.shape
    flat = inputs.reshape(-1, _EMBEDDING_DIM)
    n_tokens = flat.shape[0]
    grid = n_tokens // _BLOCK

    quantized, idx3, loss, perp = pl.pallas_call(
        functools.partial(_vq_kernel, n_tokens, grid),
        grid=(grid,),
        in_specs=[
            pl.BlockSpec((_BLOCK, _EMBEDDING_DIM), lambda i: (i, 0)),
            pl.BlockSpec((_NUM_EMBEDDINGS, _EMBEDDING_DIM), lambda i: (0, 0)),
        ],
        out_specs=[
            pl.BlockSpec((_BLOCK, _EMBEDDING_DIM), lambda i: (i, 0)),
            pl.BlockSpec((1, 1, _BLOCK), lambda i: (i, 0, 0)),
            pl.BlockSpec(memory_space=pltpu.SMEM, block_shape=(1, 1),
                         index_map=lambda i: (0, 0)),
            pl.BlockSpec(memory_space=pltpu.SMEM, block_shape=(1, 1),
                         index_map=lambda i: (0, 0)),
        ],
        out_shape=[
            jax.ShapeDtypeStruct((n_tokens, _EMBEDDING_DIM), jnp.float32),
            jax.ShapeDtypeStruct((grid, 1, _BLOCK), jnp.int32),
            jax.ShapeDtypeStruct((1, 1), jnp.float32),
            jax.ShapeDtypeStruct((1, 1), jnp.float32),
        ],
        scratch_shapes=[
            pltpu.VMEM((1, _NUM_EMBEDDINGS), jnp.float32),
            pltpu.SMEM((1, 1), jnp.float32),
        ],
    )(flat, embedding)

    quantized = quantized.reshape(input_shape)
    indices = idx3.reshape(input_shape[:-1])
    return (quantized, loss.reshape(()), indices, perp.reshape(()))
